# baseline (speedup 1.0000x reference)
import functools

import jax
import jax.numpy as jnp
from jax import lax
from jax.experimental import pallas as pl
from jax.experimental.pallas import tpu as pltpu
from jax.experimental.pallas import tpu_sc as plsc

VOCAB = 1000000
EMB = 32
B = 4096
S = 50
C = 20

NSEG = B * S
NW = 32
G = 64
IDXC = G * C
UNITS = NSEG // G
UNITS_PER_W = UNITS // NW


def _tree_sum(vals):
    while len(vals) > 1:
        nxt = [vals[i] + vals[i + 1] for i in range(0, len(vals) - 1, 2)]
        if len(vals) % 2:
            nxt.append(vals[-1])
        vals = nxt
    return vals[0]


def _make_kernel():
    mesh = plsc.VectorSubcoreMesh(core_axis_name="c", subcore_axis_name="s")

    @functools.partial(
        pl.kernel,
        mesh=mesh,
        out_type=jax.ShapeDtypeStruct((S, EMB // 8, B // 128, 8, 128), jnp.float32),
        compiler_params=pltpu.CompilerParams(
            use_tc_tiling_on_sc=False, needs_layout_passes=False
        ),
        scratch_types=[
            pltpu.VMEM((G, C), jnp.int32),
            pltpu.VMEM((IDXC,), jnp.int32),
            pltpu.VMEM((IDXC, EMB), jnp.float32),
            pltpu.VMEM((EMB // 8, 8, G), jnp.float32),
            pltpu.SemaphoreType.DMA,
        ],
    )
    def k(idx_hbm, table_hbm, out_hbm, idx2_v, idxf_v, rows_v, out_v, sem):
        wid = lax.axis_index("s") * 2 + lax.axis_index("c")
        u0 = wid * UNITS_PER_W
        lane = lax.iota(jnp.int32, 16)
        e_hi0 = lane >> 3
        e_lo0 = lane & 7
        e_hi1 = e_hi0 + 2

        def chunk_body(j, _):
            u = u0 + j
            s0 = u * G
            sv = s0 // B
            b0 = s0 - sv * B
            pltpu.sync_copy(idx_hbm.at[pl.ds(b0, G), sv, :], idx2_v)

            def flat_body(i, _):
                idxf_v[pl.ds(i * C, 16)] = idx2_v[i, pl.ds(0, 16)]
                idxf_v[pl.ds(i * C + 4, 16)] = idx2_v[i, pl.ds(4, 16)]
                return 0

            lax.fori_loop(0, G, flat_body, 0)

            handles = []
            for j128 in range(IDXC // 128):
                handles.append(
                    pltpu.async_copy(
                        table_hbm.at[idxf_v.at[pl.ds(j128 * 128, 128)]],
                        rows_v.at[pl.ds(j128 * 128, 128)],
                        sem,
                    )
                )
            for h in handles:
                h.wait()

            def seg_body(i, _):
                r0 = i * C
                acc0 = _tree_sum(
                    [rows_v[r0 + c, pl.ds(0, 16)] for c in range(C)]
                )
                acc1 = _tree_sum(
                    [rows_v[r0 + c, pl.ds(16, 16)] for c in range(C)]
                )
                bvec = jnp.full((16,), i, jnp.int32)
                plsc.store_scatter(out_v, [e_hi0, e_lo0, bvec], acc0)
                plsc.store_scatter(out_v, [e_hi1, e_lo0, bvec], acc1)
                return 0

            lax.fori_loop(0, G, seg_body, 0)
            bt = b0 // 128
            half = b0 - bt * 128
            pltpu.sync_copy(out_v, out_hbm.at[sv, :, bt, :, pl.ds(half, G)])
            return 0

        lax.fori_loop(0, UNITS_PER_W, chunk_body, 0)

    return k


_sc_kernel = _make_kernel()


def kernel(mb_t, mtd, W):
    del mtd
    out5 = _sc_kernel(mb_t.astype(jnp.int32), W)
    return out5.transpose(2, 4, 0, 1, 3).reshape(B, S, EMB)


# speedup vs baseline: 1.1665x; 1.1665x over previous
import functools

import jax
import jax.numpy as jnp
from jax import lax
from jax.experimental import pallas as pl
from jax.experimental.pallas import tpu as pltpu
from jax.experimental.pallas import tpu_sc as plsc

VOCAB = 1000000
EMB = 32
B = 4096
S = 50
C = 20

NSEG = B * S
NW = 32
G = 64
UNITS = NSEG // G
UNITS_PER_W = UNITS // NW

def _tree_sum(vals):
    while len(vals) > 1:
        nxt = [vals[i] + vals[i + 1] for i in range(0, len(vals) - 1, 2)]
        if len(vals) % 2:
            nxt.append(vals[-1])
        vals = nxt
    return vals[0]


def _make_kernel():
    mesh = plsc.VectorSubcoreMesh(core_axis_name="c", subcore_axis_name="s")

    @functools.partial(
        pl.kernel,
        mesh=mesh,
        out_type=jax.ShapeDtypeStruct((S, EMB // 8, B // 128, 8, 128), jnp.float32),
        compiler_params=pltpu.CompilerParams(
            use_tc_tiling_on_sc=False, needs_layout_passes=False
        ),
        scratch_types=[
            pltpu.VMEM((C, G), jnp.int32),
            pltpu.VMEM((C, G, EMB), jnp.float32),
            pltpu.VMEM((EMB // 8, 8, G), jnp.float32),
            pltpu.SemaphoreType.DMA,
        ],
    )
    def k(idx_hbm, table_hbm, out_hbm, idx_v, rows_v, out_v, sem):
        wid = lax.axis_index("s") * 2 + lax.axis_index("c")
        u0 = wid * UNITS_PER_W
        lane = lax.iota(jnp.int32, 16)
        e_hi0 = lane >> 3
        e_lo0 = lane & 7
        e_hi1 = e_hi0 + 2

        def chunk_body(j, _):
            u = u0 + j
            s0 = u * G
            pltpu.sync_copy(idx_hbm.at[:, pl.ds(s0, G)], idx_v)
            handles = []
            for c in range(C):
                handles.append(
                    pltpu.async_copy(
                        table_hbm.at[idx_v.at[c]], rows_v.at[c], sem
                    )
                )
            for h in handles:
                h.wait()

            def seg_body(i, _):
                acc0 = _tree_sum([rows_v[c, i, pl.ds(0, 16)] for c in range(C)])
                acc1 = _tree_sum([rows_v[c, i, pl.ds(16, 16)] for c in range(C)])
                bvec = jnp.full((16,), i, jnp.int32)
                plsc.store_scatter(out_v, [e_hi0, e_lo0, bvec], acc0)
                plsc.store_scatter(out_v, [e_hi1, e_lo0, bvec], acc1)
                return 0

            lax.fori_loop(0, G, seg_body, 0)
            sv = s0 // B
            rem = s0 - sv * B
            bt = rem // 128
            half = rem - bt * 128
            pltpu.sync_copy(out_v, out_hbm.at[sv, :, bt, :, pl.ds(half, G)])
            return 0

        lax.fori_loop(0, UNITS_PER_W, chunk_body, 0)

    return k


_sc_kernel = _make_kernel()


def kernel(mb_t, mtd, W):
    del mtd
    idx = mb_t.astype(jnp.int32).transpose(2, 1, 0).reshape(C, S * B)
    w_lin = jax.lax.optimization_barrier(W.reshape(VOCAB * EMB // 128, 128))
    out5 = _sc_kernel(idx, w_lin.reshape(VOCAB, EMB))
    return out5.transpose(2, 4, 0, 1, 3).reshape(B, S, EMB)


# speedup vs baseline: 1.2354x; 1.0591x over previous
import functools

import jax
import jax.numpy as jnp
from jax import lax
from jax.experimental import pallas as pl
from jax.experimental.pallas import tpu as pltpu
from jax.experimental.pallas import tpu_sc as plsc

VOCAB = 1000000
EMB = 32
B = 4096
S = 50
C = 20

NSEG = B * S
NW = 32
G = 64
UNITS = NSEG // G
UNITS_PER_W = UNITS // NW

_TBLK = 2048
_TGRID = pl.cdiv(VOCAB, _TBLK)
VOCAB_P = _TGRID * _TBLK


def _w_transpose_body(wt_ref, out_ref):
    x = wt_ref[...]
    for kslot in range(4):
        out_ref[:, kslot * EMB:(kslot + 1) * EMB] = (
            x[:, kslot * (_TBLK // 4):(kslot + 1) * (_TBLK // 4)].T
        )


_w_rowmajor = pl.pallas_call(
    _w_transpose_body,
    grid=(_TGRID,),
    in_specs=[pl.BlockSpec((EMB, _TBLK), lambda i: (0, i))],
    out_specs=pl.BlockSpec((_TBLK // 4, 4 * EMB), lambda i: (i, 0)),
    out_shape=jax.ShapeDtypeStruct((VOCAB_P * EMB // 128, 128), jnp.float32),
)


def _tree_sum(vals):
    while len(vals) > 1:
        nxt = [vals[i] + vals[i + 1] for i in range(0, len(vals) - 1, 2)]
        if len(vals) % 2:
            nxt.append(vals[-1])
        vals = nxt
    return vals[0]


def _make_kernel():
    mesh = plsc.VectorSubcoreMesh(core_axis_name="c", subcore_axis_name="s")

    @functools.partial(
        pl.kernel,
        mesh=mesh,
        out_type=jax.ShapeDtypeStruct((S, EMB // 8, B // 128, 8, 128), jnp.float32),
        compiler_params=pltpu.CompilerParams(
            use_tc_tiling_on_sc=False, needs_layout_passes=False
        ),
        scratch_types=[
            pltpu.VMEM((C, G), jnp.int32),
            pltpu.VMEM((C, G, EMB), jnp.float32),
            pltpu.VMEM((EMB // 8, 8, G), jnp.float32),
            pltpu.SemaphoreType.DMA,
        ],
    )
    def k(idx_hbm, table_hbm, out_hbm, idx_v, rows_v, out_v, sem):
        wid = lax.axis_index("s") * 2 + lax.axis_index("c")
        u0 = wid * UNITS_PER_W
        lane = lax.iota(jnp.int32, 16)
        e_hi0 = lane >> 3
        e_lo0 = lane & 7
        e_hi1 = e_hi0 + 2

        def chunk_body(j, _):
            u = u0 + j
            s0 = u * G
            pltpu.sync_copy(idx_hbm.at[:, pl.ds(s0, G)], idx_v)
            for c in range(C):
                for k4 in range(G // 16):
                    v = idx_v[c, pl.ds(k4 * 16, 16)]
                    idx_v[c, pl.ds(k4 * 16, 16)] = (
                        (v & (-2048)) | ((v & 511) << 2) | ((v >> 9) & 3)
                    )
            handles = []
            for c in range(C):
                handles.append(
                    pltpu.async_copy(
                        table_hbm.at[idx_v.at[c]], rows_v.at[c], sem
                    )
                )
            for h in handles:
                h.wait()

            def seg_body(i, _):
                acc0 = _tree_sum([rows_v[c, i, pl.ds(0, 16)] for c in range(C)])
                acc1 = _tree_sum([rows_v[c, i, pl.ds(16, 16)] for c in range(C)])
                bvec = jnp.full((16,), i, jnp.int32)
                plsc.store_scatter(out_v, [e_hi0, e_lo0, bvec], acc0)
                plsc.store_scatter(out_v, [e_hi1, e_lo0, bvec], acc1)
                return 0

            lax.fori_loop(0, G, seg_body, 0)
            sv = s0 // B
            rem = s0 - sv * B
            bt = rem // 128
            half = rem - bt * 128
            pltpu.sync_copy(out_v, out_hbm.at[sv, :, bt, :, pl.ds(half, G)])
            return 0

        lax.fori_loop(0, UNITS_PER_W, chunk_body, 0)

    return k


_sc_kernel = _make_kernel()


def kernel(mb_t, mtd, W):
    del mtd
    idx = mb_t.astype(jnp.int32).transpose(2, 1, 0).reshape(C, S * B)
    w_st = _w_rowmajor(W.T)
    out5 = _sc_kernel(idx, w_st.reshape(VOCAB_P, EMB))
    return out5.transpose(2, 4, 0, 1, 3).reshape(B, S, EMB)


# speedup vs baseline: 1.5391x; 1.2458x over previous
import functools

import jax
import jax.numpy as jnp
from jax import lax
from jax.experimental import pallas as pl
from jax.experimental.pallas import tpu as pltpu
from jax.experimental.pallas import tpu_sc as plsc

VOCAB = 1000000
EMB = 32
B = 4096
S = 50
C = 20

NSEG = B * S
NW = 32
G = 64
UNITS = NSEG // G
UNITS_PER_W = UNITS // NW

_TBLK = 2048
_TGRID = pl.cdiv(VOCAB, _TBLK)
VOCAB_P = _TGRID * _TBLK


def _w_transpose_body(wt_ref, out_ref):
    x = wt_ref[...]
    for kslot in range(4):
        out_ref[:, kslot * EMB:(kslot + 1) * EMB] = (
            x[:, kslot * (_TBLK // 4):(kslot + 1) * (_TBLK // 4)].T
        )


_w_rowmajor = pl.pallas_call(
    _w_transpose_body,
    grid=(_TGRID,),
    in_specs=[pl.BlockSpec((EMB, _TBLK), lambda i: (0, i))],
    out_specs=pl.BlockSpec((_TBLK // 4, 4 * EMB), lambda i: (i, 0)),
    out_shape=jax.ShapeDtypeStruct((VOCAB_P * EMB // 128, 128), jnp.float32),
)


def _tree_sum(vals):
    while len(vals) > 1:
        nxt = [vals[i] + vals[i + 1] for i in range(0, len(vals) - 1, 2)]
        if len(vals) % 2:
            nxt.append(vals[-1])
        vals = nxt
    return vals[0]


def _make_kernel():
    mesh = plsc.VectorSubcoreMesh(core_axis_name="c", subcore_axis_name="s")

    @functools.partial(
        pl.kernel,
        mesh=mesh,
        out_type=jax.ShapeDtypeStruct((S, EMB // 8, B // 128, 8, 128), jnp.float32),
        compiler_params=pltpu.CompilerParams(
            use_tc_tiling_on_sc=False, needs_layout_passes=False
        ),
        scratch_types=[
            pltpu.VMEM((C, G), jnp.int32),
            pltpu.VMEM((C, G), jnp.int32),
            pltpu.VMEM((C, G, EMB), jnp.float32),
            pltpu.VMEM((C, G, EMB), jnp.float32),
            pltpu.VMEM((EMB // 8, 8, G), jnp.float32),
            pltpu.VMEM((EMB // 8, 8, G), jnp.float32),
            pltpu.SemaphoreType.DMA,
            pltpu.SemaphoreType.DMA,
        ],
    )
    def k(idx_hbm, table_hbm, out_hbm, idx_a, idx_b, rows_a, rows_b,
          out_a, out_b, sem_a, sem_b):
        wid = lax.axis_index("s") * 2 + lax.axis_index("c")
        u0 = wid * UNITS_PER_W
        lane = lax.iota(jnp.int32, 16)
        e_hi0 = lane >> 3
        e_lo0 = lane & 7
        e_hi1 = e_hi0 + 2

        def start(j, iv, rv, sem):
            s0 = (u0 + j) * G
            pltpu.sync_copy(idx_hbm.at[:, pl.ds(s0, G)], iv)
            for c in range(C):
                for k4 in range(G // 16):
                    v = iv[c, pl.ds(k4 * 16, 16)]
                    iv[c, pl.ds(k4 * 16, 16)] = (
                        (v & (-2048)) | ((v & 511) << 2) | ((v >> 9) & 3)
                    )
            for c in range(C):
                pltpu.async_copy(table_hbm.at[iv.at[c]], rv.at[c], sem)

        def finish(j, iv, rv, ov, sem):
            for c in range(C):
                pltpu.make_async_copy(
                    table_hbm.at[iv.at[c]], rv.at[c], sem
                ).wait()

            def seg_body(i, _):
                acc0 = _tree_sum([rv[c, i, pl.ds(0, 16)] for c in range(C)])
                acc1 = _tree_sum([rv[c, i, pl.ds(16, 16)] for c in range(C)])
                bvec = jnp.full((16,), i, jnp.int32)
                plsc.store_scatter(ov, [e_hi0, e_lo0, bvec], acc0)
                plsc.store_scatter(ov, [e_hi1, e_lo0, bvec], acc1)
                return 0

            lax.fori_loop(0, G, seg_body, 0)
            s0 = (u0 + j) * G
            sv = s0 // B
            rem = s0 - sv * B
            bt = rem // 128
            half = rem - bt * 128
            pltpu.sync_copy(ov, out_hbm.at[sv, :, bt, :, pl.ds(half, G)])

        start(0, idx_a, rows_a, sem_a)

        def pair_body(i, _):
            u = 2 * i
            start(u + 1, idx_b, rows_b, sem_b)
            finish(u, idx_a, rows_a, out_a, sem_a)
            start(u + 2, idx_a, rows_a, sem_a)
            finish(u + 1, idx_b, rows_b, out_b, sem_b)
            return 0

        lax.fori_loop(0, UNITS_PER_W // 2 - 1, pair_body, 0)
        start(UNITS_PER_W - 1, idx_b, rows_b, sem_b)
        finish(UNITS_PER_W - 2, idx_a, rows_a, out_a, sem_a)
        finish(UNITS_PER_W - 1, idx_b, rows_b, out_b, sem_b)

    return k


_sc_kernel = _make_kernel()


def kernel(mb_t, mtd, W):
    del mtd
    idx = mb_t.astype(jnp.int32).transpose(2, 1, 0).reshape(C, S * B)
    w_st = _w_rowmajor(W.T)
    out5 = _sc_kernel(idx, w_st.reshape(VOCAB_P, EMB))
    return out5.transpose(2, 4, 0, 1, 3).reshape(B, S, EMB)


# speedup vs baseline: 1.9072x; 1.2392x over previous
import functools

import jax
import jax.numpy as jnp
from jax import lax
from jax.experimental import pallas as pl
from jax.experimental.pallas import tpu as pltpu
from jax.experimental.pallas import tpu_sc as plsc

VOCAB = 1000000
EMB = 32
B = 4096
S = 50
C = 20

NSEG = B * S
NW = 32
G = 64
UNITS = NSEG // G
UNITS_PER_W = UNITS // NW

_TBLK = 8192
_TGRID = pl.cdiv(VOCAB, _TBLK)
VOCAB_P = _TGRID * _TBLK
_KP = _TBLK // 4
_KPLOG = _KP.bit_length() - 1


def _w_transpose_body(wt_ref, out_ref):
    x = wt_ref[...]
    out_ref[...] = jnp.concatenate(
        [x[:, kslot * _KP:(kslot + 1) * _KP].T for kslot in range(4)],
        axis=1,
    )


_w_rowmajor = pl.pallas_call(
    _w_transpose_body,
    grid=(_TGRID,),
    in_specs=[pl.BlockSpec((EMB, _TBLK), lambda i: (0, i))],
    out_specs=pl.BlockSpec((_TBLK // 4, 4 * EMB), lambda i: (i, 0)),
    out_shape=jax.ShapeDtypeStruct((VOCAB_P * EMB // 128, 128), jnp.float32),
)


def _tree_sum(vals):
    while len(vals) > 1:
        nxt = [vals[i] + vals[i + 1] for i in range(0, len(vals) - 1, 2)]
        if len(vals) % 2:
            nxt.append(vals[-1])
        vals = nxt
    return vals[0]


def _make_kernel():
    mesh = plsc.VectorSubcoreMesh(core_axis_name="c", subcore_axis_name="s")

    @functools.partial(
        pl.kernel,
        mesh=mesh,
        out_type=jax.ShapeDtypeStruct((S, EMB // 8, B // 128, 8, 128), jnp.float32),
        compiler_params=pltpu.CompilerParams(
            use_tc_tiling_on_sc=False, needs_layout_passes=False
        ),
        scratch_types=[
            pltpu.VMEM((C, G), jnp.int32),
            pltpu.VMEM((C, G), jnp.int32),
            pltpu.VMEM((C, G, EMB), jnp.float32),
            pltpu.VMEM((C, G, EMB), jnp.float32),
            pltpu.VMEM((EMB // 8, 8, G), jnp.float32),
            pltpu.VMEM((EMB // 8, 8, G), jnp.float32),
            pltpu.SemaphoreType.DMA,
            pltpu.SemaphoreType.DMA,
        ],
    )
    def k(idx_hbm, table_hbm, out_hbm, idx_a, idx_b, rows_a, rows_b,
          out_a, out_b, sem_a, sem_b):
        wid = lax.axis_index("s") * 2 + lax.axis_index("c")
        u0 = wid * UNITS_PER_W
        lane = lax.iota(jnp.int32, 16)
        e_hi0 = lane >> 3
        e_lo0 = lane & 7
        e_hi1 = e_hi0 + 2

        def start(j, iv, rv, sem):
            s0 = (u0 + j) * G
            pltpu.sync_copy(idx_hbm.at[:, pl.ds(s0, G)], iv)
            for c in range(C):
                for k4 in range(G // 16):
                    v = iv[c, pl.ds(k4 * 16, 16)]
                    iv[c, pl.ds(k4 * 16, 16)] = (
                        (v & (-_TBLK))
                        | ((v & (_KP - 1)) << 2)
                        | ((v >> _KPLOG) & 3)
                    )
            for c in range(C):
                pltpu.async_copy(table_hbm.at[iv.at[c]], rv.at[c], sem)

        def finish(j, iv, rv, ov, sem):
            for c in range(C):
                pltpu.make_async_copy(
                    table_hbm.at[iv.at[c]], rv.at[c], sem
                ).wait()

            def seg_body(i, _):
                acc0 = _tree_sum([rv[c, i, pl.ds(0, 16)] for c in range(C)])
                acc1 = _tree_sum([rv[c, i, pl.ds(16, 16)] for c in range(C)])
                bvec = jnp.full((16,), i, jnp.int32)
                plsc.store_scatter(ov, [e_hi0, e_lo0, bvec], acc0)
                plsc.store_scatter(ov, [e_hi1, e_lo0, bvec], acc1)
                return 0

            lax.fori_loop(0, G, seg_body, 0)
            s0 = (u0 + j) * G
            sv = s0 // B
            rem = s0 - sv * B
            bt = rem // 128
            half = rem - bt * 128
            pltpu.sync_copy(ov, out_hbm.at[sv, :, bt, :, pl.ds(half, G)])

        start(0, idx_a, rows_a, sem_a)

        def pair_body(i, _):
            u = 2 * i
            start(u + 1, idx_b, rows_b, sem_b)
            finish(u, idx_a, rows_a, out_a, sem_a)
            start(u + 2, idx_a, rows_a, sem_a)
            finish(u + 1, idx_b, rows_b, out_b, sem_b)
            return 0

        lax.fori_loop(0, UNITS_PER_W // 2 - 1, pair_body, 0)
        start(UNITS_PER_W - 1, idx_b, rows_b, sem_b)
        finish(UNITS_PER_W - 2, idx_a, rows_a, out_a, sem_a)
        finish(UNITS_PER_W - 1, idx_b, rows_b, out_b, sem_b)

    return k


_sc_kernel = _make_kernel()


def kernel(mb_t, mtd, W):
    del mtd
    idx = mb_t.astype(jnp.int32).transpose(2, 1, 0).reshape(C, S * B)
    w_st = _w_rowmajor(W.T)
    out5 = _sc_kernel(idx, w_st.reshape(VOCAB_P, EMB))
    return out5.transpose(2, 4, 0, 1, 3).reshape(B, S, EMB)


# speedup vs baseline: 1.9267x; 1.0102x over previous
import functools

import jax
import jax.numpy as jnp
from jax import lax
from jax.experimental import pallas as pl
from jax.experimental.pallas import tpu as pltpu
from jax.experimental.pallas import tpu_sc as plsc

VOCAB = 1000000
EMB = 32
B = 4096
S = 50
C = 20

NSEG = B * S
NW = 32
G = 64
UNITS = NSEG // G
UNITS_PER_W = UNITS // NW

_TBLK = 16384
_TGRID = pl.cdiv(VOCAB, _TBLK)
VOCAB_P = _TGRID * _TBLK
_KP = _TBLK // 4
_KPLOG = _KP.bit_length() - 1


def _w_transpose_body(wt_ref, out_ref):
    x = wt_ref[...]
    out_ref[...] = jnp.concatenate(
        [x[:, kslot * _KP:(kslot + 1) * _KP].T for kslot in range(4)],
        axis=1,
    )


_w_rowmajor = pl.pallas_call(
    _w_transpose_body,
    grid=(_TGRID,),
    in_specs=[pl.BlockSpec((EMB, _TBLK), lambda i: (0, i))],
    out_specs=pl.BlockSpec((_TBLK // 4, 4 * EMB), lambda i: (i, 0)),
    out_shape=jax.ShapeDtypeStruct((VOCAB_P * EMB // 128, 128), jnp.float32),
)


def _tree_sum(vals):
    while len(vals) > 1:
        nxt = [vals[i] + vals[i + 1] for i in range(0, len(vals) - 1, 2)]
        if len(vals) % 2:
            nxt.append(vals[-1])
        vals = nxt
    return vals[0]


def _make_kernel():
    mesh = plsc.VectorSubcoreMesh(core_axis_name="c", subcore_axis_name="s")

    @functools.partial(
        pl.kernel,
        mesh=mesh,
        out_type=jax.ShapeDtypeStruct((S, EMB // 8, B // 128, 8, 128), jnp.float32),
        compiler_params=pltpu.CompilerParams(
            use_tc_tiling_on_sc=False, needs_layout_passes=False
        ),
        scratch_types=[
            pltpu.VMEM((C, G), jnp.int32),
            pltpu.VMEM((C, G), jnp.int32),
            pltpu.VMEM((C // 2, 2 * G), jnp.int32),
            pltpu.VMEM((C // 2, 2 * G), jnp.int32),
            pltpu.VMEM((C // 2, 2 * G, EMB), jnp.float32),
            pltpu.VMEM((C // 2, 2 * G, EMB), jnp.float32),
            pltpu.VMEM((EMB // 8, 8, G), jnp.float32),
            pltpu.VMEM((EMB // 8, 8, G), jnp.float32),
            pltpu.SemaphoreType.DMA,
            pltpu.SemaphoreType.DMA,
        ],
    )
    def k(idx_hbm, table_hbm, out_hbm, idx_a, idx_b, idx2_a, idx2_b,
          rows_a, rows_b, out_a, out_b, sem_a, sem_b):
        wid = lax.axis_index("s") * 2 + lax.axis_index("c")
        u0 = wid * UNITS_PER_W
        lane = lax.iota(jnp.int32, 16)
        e_hi0 = lane >> 3
        e_lo0 = lane & 7
        e_hi1 = e_hi0 + 2

        def start(j, iv, iv2, rv, sem):
            s0 = (u0 + j) * G
            pltpu.sync_copy(idx_hbm.at[:, pl.ds(s0, G)], iv)
            for c in range(C):
                for k4 in range(G // 16):
                    v = iv[c, pl.ds(k4 * 16, 16)]
                    iv2[c // 2, pl.ds((c % 2) * G + k4 * 16, 16)] = (
                        (v & (-_TBLK))
                        | ((v & (_KP - 1)) << 2)
                        | ((v >> _KPLOG) & 3)
                    )
            for c2 in range(C // 2):
                pltpu.async_copy(table_hbm.at[iv2.at[c2]], rv.at[c2], sem)

        def finish(j, iv2, rv, ov, sem):
            for c2 in range(C // 2):
                pltpu.make_async_copy(
                    table_hbm.at[iv2.at[c2]], rv.at[c2], sem
                ).wait()

            def seg_body(i, _):
                acc0 = _tree_sum(
                    [rv[c // 2, (c % 2) * G + i, pl.ds(0, 16)]
                     for c in range(C)]
                )
                acc1 = _tree_sum(
                    [rv[c // 2, (c % 2) * G + i, pl.ds(16, 16)]
                     for c in range(C)]
                )
                bvec = jnp.full((16,), i, jnp.int32)
                plsc.store_scatter(ov, [e_hi0, e_lo0, bvec], acc0)
                plsc.store_scatter(ov, [e_hi1, e_lo0, bvec], acc1)
                return 0

            lax.fori_loop(0, G, seg_body, 0)
            s0 = (u0 + j) * G
            sv = s0 // B
            rem = s0 - sv * B
            bt = rem // 128
            half = rem - bt * 128
            pltpu.sync_copy(ov, out_hbm.at[sv, :, bt, :, pl.ds(half, G)])

        start(0, idx_a, idx2_a, rows_a, sem_a)

        def pair_body(i, _):
            u = 2 * i
            start(u + 1, idx_b, idx2_b, rows_b, sem_b)
            finish(u, idx2_a, rows_a, out_a, sem_a)
            start(u + 2, idx_a, idx2_a, rows_a, sem_a)
            finish(u + 1, idx2_b, rows_b, out_b, sem_b)
            return 0

        lax.fori_loop(0, UNITS_PER_W // 2 - 1, pair_body, 0)
        start(UNITS_PER_W - 1, idx_b, idx2_b, rows_b, sem_b)
        finish(UNITS_PER_W - 2, idx2_a, rows_a, out_a, sem_a)
        finish(UNITS_PER_W - 1, idx2_b, rows_b, out_b, sem_b)

    return k


_sc_kernel = _make_kernel()


def kernel(mb_t, mtd, W):
    del mtd
    idx = mb_t.astype(jnp.int32).transpose(2, 1, 0).reshape(C, S * B)
    w_st = _w_rowmajor(W.T)
    out5 = _sc_kernel(idx, w_st.reshape(VOCAB_P, EMB))
    return out5.transpose(2, 4, 0, 1, 3).reshape(B, S, EMB)


# speedup vs baseline: 2.1328x; 1.1070x over previous
import functools

import jax
import jax.numpy as jnp
from jax import lax
from jax.experimental import pallas as pl
from jax.experimental.pallas import tpu as pltpu
from jax.experimental.pallas import tpu_sc as plsc

VOCAB = 1000000
EMB = 32
B = 4096
S = 50
C = 20

NSEG = B * S
NW = 32
G = 64
UNITS = NSEG // G
UNITS_PER_W = UNITS // NW

_TBLK = 16384
_TGRID = pl.cdiv(VOCAB, _TBLK)
VOCAB_P = _TGRID * _TBLK
_KP = _TBLK // 4
_KPLOG = _KP.bit_length() - 1


def _w_transpose_body(wt_ref, out_ref):
    x = wt_ref[...]
    out_ref[...] = jnp.concatenate(
        [x[:, kslot * _KP:(kslot + 1) * _KP].T for kslot in range(4)],
        axis=1,
    )


_w_rowmajor = pl.pallas_call(
    _w_transpose_body,
    grid=(_TGRID,),
    in_specs=[pl.BlockSpec((EMB, _TBLK), lambda i: (0, i))],
    out_specs=pl.BlockSpec((_TBLK // 4, 4 * EMB), lambda i: (i, 0)),
    out_shape=jax.ShapeDtypeStruct((VOCAB_P * EMB // 128, 128), jnp.float32),
)


def _tree_sum(vals):
    while len(vals) > 1:
        nxt = [vals[i] + vals[i + 1] for i in range(0, len(vals) - 1, 2)]
        if len(vals) % 2:
            nxt.append(vals[-1])
        vals = nxt
    return vals[0]


def _make_kernel():
    mesh = plsc.VectorSubcoreMesh(core_axis_name="c", subcore_axis_name="s")

    @functools.partial(
        pl.kernel,
        mesh=mesh,
        out_type=jax.ShapeDtypeStruct((S, EMB // 8, B // 128, 8, 128), jnp.float32),
        compiler_params=pltpu.CompilerParams(
            use_tc_tiling_on_sc=False, needs_layout_passes=False
        ),
        scratch_types=[
            pltpu.VMEM((C, G), jnp.int32),
            pltpu.VMEM((C, G), jnp.int32),
            pltpu.VMEM((C // 2, 2 * G), jnp.int32),
            pltpu.VMEM((C // 2, 2 * G), jnp.int32),
            pltpu.VMEM((C // 2, 2 * G, EMB), jnp.float32),
            pltpu.VMEM((C // 2, 2 * G, EMB), jnp.float32),
            pltpu.VMEM((EMB // 8, 8, G), jnp.float32),
            pltpu.VMEM((EMB // 8, 8, G), jnp.float32),
            pltpu.SemaphoreType.DMA,
            pltpu.SemaphoreType.DMA,
            pltpu.SemaphoreType.DMA,
            pltpu.SemaphoreType.DMA,
        ],
    )
    def k(idx_hbm, table_hbm, out_hbm, idx_a, idx_b, idx2_a, idx2_b,
          rows_a, rows_b, out_a, out_b, sem_a, sem_b, isem_a, isem_b):
        wid = lax.axis_index("s") * 2 + lax.axis_index("c")
        u0 = wid * UNITS_PER_W
        lane = lax.iota(jnp.int32, 16)
        e_hi0 = lane >> 3
        e_lo0 = lane & 7
        e_hi1 = e_hi0 + 2

        def prefetch(j, iv, isem):
            s0 = jnp.minimum((u0 + j) * G, NSEG - G)
            pltpu.async_copy(idx_hbm.at[:, pl.ds(s0, G)], iv, isem)

        def start(j, iv, iv2, rv, sem, isem):
            s0 = (u0 + j) * G
            pltpu.make_async_copy(
                idx_hbm.at[:, pl.ds(jnp.minimum(s0, NSEG - G), G)], iv, isem
            ).wait()
            for c in range(C):
                for k4 in range(G // 16):
                    v = iv[c, pl.ds(k4 * 16, 16)]
                    iv2[c // 2, pl.ds((c % 2) * G + k4 * 16, 16)] = (
                        (v & (-_TBLK))
                        | ((v & (_KP - 1)) << 2)
                        | ((v >> _KPLOG) & 3)
                    )
            for c2 in range(C // 2):
                pltpu.async_copy(table_hbm.at[iv2.at[c2]], rv.at[c2], sem)
            prefetch(j + 2, iv, isem)

        def finish(j, iv2, rv, ov, sem):
            for c2 in range(C // 2):
                pltpu.make_async_copy(
                    table_hbm.at[iv2.at[c2]], rv.at[c2], sem
                ).wait()

            def seg_body(i, _):
                acc0 = _tree_sum(
                    [rv[c // 2, (c % 2) * G + i, pl.ds(0, 16)]
                     for c in range(C)]
                )
                acc1 = _tree_sum(
                    [rv[c // 2, (c % 2) * G + i, pl.ds(16, 16)]
                     for c in range(C)]
                )
                bvec = jnp.full((16,), i, jnp.int32)
                plsc.store_scatter(ov, [e_hi0, e_lo0, bvec], acc0)
                plsc.store_scatter(ov, [e_hi1, e_lo0, bvec], acc1)
                return 0

            lax.fori_loop(0, G, seg_body, 0)
            s0 = (u0 + j) * G
            sv = s0 // B
            rem = s0 - sv * B
            bt = rem // 128
            half = rem - bt * 128
            pltpu.sync_copy(ov, out_hbm.at[sv, :, bt, :, pl.ds(half, G)])

        prefetch(0, idx_a, isem_a)
        prefetch(1, idx_b, isem_b)
        start(0, idx_a, idx2_a, rows_a, sem_a, isem_a)

        def pair_body(i, _):
            u = 2 * i
            start(u + 1, idx_b, idx2_b, rows_b, sem_b, isem_b)
            finish(u, idx2_a, rows_a, out_a, sem_a)
            start(u + 2, idx_a, idx2_a, rows_a, sem_a, isem_a)
            finish(u + 1, idx2_b, rows_b, out_b, sem_b)
            return 0

        lax.fori_loop(0, UNITS_PER_W // 2 - 1, pair_body, 0)
        start(UNITS_PER_W - 1, idx_b, idx2_b, rows_b, sem_b, isem_b)
        finish(UNITS_PER_W - 2, idx2_a, rows_a, out_a, sem_a)
        finish(UNITS_PER_W - 1, idx2_b, rows_b, out_b, sem_b)
        pltpu.make_async_copy(
            idx_hbm.at[:, pl.ds(NSEG - G, G)], idx_a, isem_a
        ).wait()
        pltpu.make_async_copy(
            idx_hbm.at[:, pl.ds(NSEG - G, G)], idx_b, isem_b
        ).wait()

    return k


_sc_kernel = _make_kernel()


def kernel(mb_t, mtd, W):
    del mtd
    idx = mb_t.astype(jnp.int32).transpose(2, 1, 0).reshape(C, S * B)
    w_st = _w_rowmajor(W.T)
    out5 = _sc_kernel(idx, w_st.reshape(VOCAB_P, EMB))
    return out5.transpose(2, 4, 0, 1, 3).reshape(B, S, EMB)


# speedup vs baseline: 2.1574x; 1.0115x over previous
import functools

import jax
import jax.numpy as jnp
from jax import lax
from jax.experimental import pallas as pl
from jax.experimental.pallas import tpu as pltpu
from jax.experimental.pallas import tpu_sc as plsc

VOCAB = 1000000
EMB = 32
B = 4096
S = 50
C = 20

NSEG = B * S
NW = 32
G = 64
UNITS = NSEG // G
UNITS_PER_W = UNITS // NW

_TBLK = 16384
_TGRID = pl.cdiv(VOCAB, _TBLK)
VOCAB_P = _TGRID * _TBLK
_KP = _TBLK // 4
_KPLOG = _KP.bit_length() - 1


def _w_transpose_body(wt_ref, out_ref):
    x = wt_ref[...]
    out_ref[...] = jnp.concatenate(
        [x[:, kslot * _KP:(kslot + 1) * _KP].T for kslot in range(4)],
        axis=1,
    )


_w_rowmajor = pl.pallas_call(
    _w_transpose_body,
    grid=(_TGRID,),
    in_specs=[pl.BlockSpec((EMB, _TBLK), lambda i: (0, i))],
    out_specs=pl.BlockSpec((_TBLK // 4, 4 * EMB), lambda i: (i, 0)),
    out_shape=jax.ShapeDtypeStruct((VOCAB_P * EMB // 128, 128), jnp.float32),
)


def _tree_sum(vals):
    while len(vals) > 1:
        nxt = [vals[i] + vals[i + 1] for i in range(0, len(vals) - 1, 2)]
        if len(vals) % 2:
            nxt.append(vals[-1])
        vals = nxt
    return vals[0]


def _make_kernel():
    mesh = plsc.VectorSubcoreMesh(core_axis_name="c", subcore_axis_name="s")

    @functools.partial(
        pl.kernel,
        mesh=mesh,
        out_type=jax.ShapeDtypeStruct((S, EMB // 8, B // 128, 8, 128), jnp.float32),
        compiler_params=pltpu.CompilerParams(
            use_tc_tiling_on_sc=False, needs_layout_passes=False
        ),
        scratch_types=[
            pltpu.VMEM((C, G), jnp.int32),
            pltpu.VMEM((C, G), jnp.int32),
            pltpu.VMEM((C // 2, 2 * G), jnp.int32),
            pltpu.VMEM((C // 2, 2 * G), jnp.int32),
            pltpu.VMEM((C // 2, 2 * G, EMB), jnp.float32),
            pltpu.VMEM((C // 2, 2 * G, EMB), jnp.float32),
            pltpu.VMEM((EMB // 8, 8, G), jnp.float32),
            pltpu.VMEM((EMB // 8, 8, G), jnp.float32),
            pltpu.SemaphoreType.DMA,
            pltpu.SemaphoreType.DMA,
            pltpu.SemaphoreType.DMA,
            pltpu.SemaphoreType.DMA,
            pltpu.SemaphoreType.DMA,
            pltpu.SemaphoreType.DMA,
        ],
    )
    def k(idx_hbm, table_hbm, out_hbm, idx_a, idx_b, idx2_a, idx2_b,
          rows_a, rows_b, out_a, out_b, sem_a, sem_b, isem_a, isem_b,
          osem_a, osem_b):
        wid = lax.axis_index("s") * 2 + lax.axis_index("c")
        u0 = wid * UNITS_PER_W
        lane = lax.iota(jnp.int32, 16)
        e_hi0 = lane >> 3
        e_lo0 = lane & 7
        e_hi1 = e_hi0 + 2

        def prefetch(j, iv, isem):
            s0 = jnp.minimum((u0 + j) * G, NSEG - G)
            pltpu.async_copy(idx_hbm.at[:, pl.ds(s0, G)], iv, isem)

        def start(j, iv, iv2, rv, sem, isem):
            s0 = (u0 + j) * G
            pltpu.make_async_copy(
                idx_hbm.at[:, pl.ds(jnp.minimum(s0, NSEG - G), G)], iv, isem
            ).wait()
            for c in range(C):
                for k4 in range(G // 16):
                    v = iv[c, pl.ds(k4 * 16, 16)]
                    iv2[c // 2, pl.ds((c % 2) * G + k4 * 16, 16)] = (
                        (v & (-_TBLK))
                        | ((v & (_KP - 1)) << 2)
                        | ((v >> _KPLOG) & 3)
                    )
            for c2 in range(C // 2):
                pltpu.async_copy(table_hbm.at[iv2.at[c2]], rv.at[c2], sem)
            prefetch(j + 2, iv, isem)

        def out_dst(j):
            s0 = (u0 + j) * G
            sv = s0 // B
            rem = s0 - sv * B
            bt = rem // 128
            half = rem - bt * 128
            return out_hbm.at[sv, :, bt, :, pl.ds(half, G)]

        def finish(j, iv2, rv, ov, sem, osem):
            for c2 in range(C // 2):
                pltpu.make_async_copy(
                    table_hbm.at[iv2.at[c2]], rv.at[c2], sem
                ).wait()
            pltpu.make_async_copy(ov, out_dst(j), osem).wait()

            def seg_body(i, _):
                acc0 = _tree_sum(
                    [rv[c // 2, (c % 2) * G + i, pl.ds(0, 16)]
                     for c in range(C)]
                )
                acc1 = _tree_sum(
                    [rv[c // 2, (c % 2) * G + i, pl.ds(16, 16)]
                     for c in range(C)]
                )
                bvec = jnp.full((16,), i, jnp.int32)
                plsc.store_scatter(ov, [e_hi0, e_lo0, bvec], acc0)
                plsc.store_scatter(ov, [e_hi1, e_lo0, bvec], acc1)
                return 0

            lax.fori_loop(0, G, seg_body, 0)
            pltpu.async_copy(ov, out_dst(j), osem)

        prefetch(0, idx_a, isem_a)
        prefetch(1, idx_b, isem_b)
        pltpu.async_copy(out_a, out_dst(0), osem_a)
        pltpu.async_copy(out_b, out_dst(1), osem_b)
        start(0, idx_a, idx2_a, rows_a, sem_a, isem_a)

        def pair_body(i, _):
            u = 2 * i
            start(u + 1, idx_b, idx2_b, rows_b, sem_b, isem_b)
            finish(u, idx2_a, rows_a, out_a, sem_a, osem_a)
            start(u + 2, idx_a, idx2_a, rows_a, sem_a, isem_a)
            finish(u + 1, idx2_b, rows_b, out_b, sem_b, osem_b)
            return 0

        lax.fori_loop(0, UNITS_PER_W // 2 - 1, pair_body, 0)
        start(UNITS_PER_W - 1, idx_b, idx2_b, rows_b, sem_b, isem_b)
        finish(UNITS_PER_W - 2, idx2_a, rows_a, out_a, sem_a, osem_a)
        finish(UNITS_PER_W - 1, idx2_b, rows_b, out_b, sem_b, osem_b)
        pltpu.make_async_copy(out_a, out_dst(0), osem_a).wait()
        pltpu.make_async_copy(out_b, out_dst(1), osem_b).wait()
        pltpu.make_async_copy(
            idx_hbm.at[:, pl.ds(NSEG - G, G)], idx_a, isem_a
        ).wait()
        pltpu.make_async_copy(
            idx_hbm.at[:, pl.ds(NSEG - G, G)], idx_b, isem_b
        ).wait()

    return k


_sc_kernel = _make_kernel()


def kernel(mb_t, mtd, W):
    del mtd
    idx = mb_t.astype(jnp.int32).transpose(2, 1, 0).reshape(C, S * B)
    w_st = _w_rowmajor(W.T)
    out5 = _sc_kernel(idx, w_st.reshape(VOCAB_P, EMB))
    return out5.transpose(2, 4, 0, 1, 3).reshape(B, S, EMB)


# speedup vs baseline: 2.1658x; 1.0039x over previous
import functools

import jax
import jax.numpy as jnp
from jax import lax
from jax.experimental import pallas as pl
from jax.experimental.pallas import tpu as pltpu
from jax.experimental.pallas import tpu_sc as plsc

VOCAB = 1000000
EMB = 32
B = 4096
S = 50
C = 20

NSEG = B * S
NW = 32
G = 64
UNITS = NSEG // G
UNITS_PER_W = UNITS // NW

_TBLK = 32768
_TGRID = pl.cdiv(VOCAB, _TBLK)
VOCAB_P = _TGRID * _TBLK
_KP = _TBLK // 4
_KPLOG = _KP.bit_length() - 1


def _w_transpose_body(wt_ref, out_ref):
    x = wt_ref[...]
    out_ref[...] = jnp.concatenate(
        [x[:, kslot * _KP:(kslot + 1) * _KP].T for kslot in range(4)],
        axis=1,
    )


_w_rowmajor = pl.pallas_call(
    _w_transpose_body,
    grid=(_TGRID,),
    in_specs=[pl.BlockSpec((EMB, _TBLK), lambda i: (0, i))],
    out_specs=pl.BlockSpec((_TBLK // 4, 4 * EMB), lambda i: (i, 0)),
    out_shape=jax.ShapeDtypeStruct((VOCAB_P * EMB // 128, 128), jnp.float32),
)


def _tree_sum(vals):
    while len(vals) > 1:
        nxt = [vals[i] + vals[i + 1] for i in range(0, len(vals) - 1, 2)]
        if len(vals) % 2:
            nxt.append(vals[-1])
        vals = nxt
    return vals[0]


def _make_kernel():
    mesh = plsc.VectorSubcoreMesh(core_axis_name="c", subcore_axis_name="s")

    @functools.partial(
        pl.kernel,
        mesh=mesh,
        out_type=jax.ShapeDtypeStruct((S, EMB // 8, B // 128, 8, 128), jnp.float32),
        compiler_params=pltpu.CompilerParams(
            use_tc_tiling_on_sc=False, needs_layout_passes=False
        ),
        scratch_types=[
            pltpu.VMEM((C, G), jnp.int32),
            pltpu.VMEM((C, G), jnp.int32),
            pltpu.VMEM((C // 2, 2 * G), jnp.int32),
            pltpu.VMEM((C // 2, 2 * G), jnp.int32),
            pltpu.VMEM((C // 2, 2 * G, EMB), jnp.float32),
            pltpu.VMEM((C // 2, 2 * G, EMB), jnp.float32),
            pltpu.VMEM((EMB // 8, 8, G), jnp.float32),
            pltpu.VMEM((EMB // 8, 8, G), jnp.float32),
            pltpu.SemaphoreType.DMA,
            pltpu.SemaphoreType.DMA,
            pltpu.SemaphoreType.DMA,
            pltpu.SemaphoreType.DMA,
            pltpu.SemaphoreType.DMA,
            pltpu.SemaphoreType.DMA,
        ],
    )
    def k(idx_hbm, table_hbm, out_hbm, idx_a, idx_b, idx2_a, idx2_b,
          rows_a, rows_b, out_a, out_b, sem_a, sem_b, isem_a, isem_b,
          osem_a, osem_b):
        wid = lax.axis_index("s") * 2 + lax.axis_index("c")
        u0 = wid * UNITS_PER_W
        lane = lax.iota(jnp.int32, 16)
        e_hi0 = lane >> 3
        e_lo0 = lane & 7
        e_hi1 = e_hi0 + 2

        def prefetch(j, iv, isem):
            s0 = jnp.minimum((u0 + j) * G, NSEG - G)
            pltpu.async_copy(idx_hbm.at[:, pl.ds(s0, G)], iv, isem)

        def start(j, iv, iv2, rv, sem, isem):
            s0 = (u0 + j) * G
            pltpu.make_async_copy(
                idx_hbm.at[:, pl.ds(jnp.minimum(s0, NSEG - G), G)], iv, isem
            ).wait()
            for c in range(C):
                for k4 in range(G // 16):
                    v = iv[c, pl.ds(k4 * 16, 16)]
                    iv2[c // 2, pl.ds((c % 2) * G + k4 * 16, 16)] = (
                        (v & (-_TBLK))
                        | ((v & (_KP - 1)) << 2)
                        | ((v >> _KPLOG) & 3)
                    )
            for c2 in range(C // 2):
                pltpu.async_copy(table_hbm.at[iv2.at[c2]], rv.at[c2], sem)
            prefetch(j + 2, iv, isem)

        def out_dst(j):
            s0 = (u0 + j) * G
            sv = s0 // B
            rem = s0 - sv * B
            bt = rem // 128
            half = rem - bt * 128
            return out_hbm.at[sv, :, bt, :, pl.ds(half, G)]

        def finish(j, iv2, rv, ov, sem, osem):
            for c2 in range(C // 2):
                pltpu.make_async_copy(
                    table_hbm.at[iv2.at[c2]], rv.at[c2], sem
                ).wait()
            pltpu.make_async_copy(ov, out_dst(j), osem).wait()

            def seg_body(i, _):
                acc0 = _tree_sum(
                    [rv[c // 2, (c % 2) * G + i, pl.ds(0, 16)]
                     for c in range(C)]
                )
                acc1 = _tree_sum(
                    [rv[c // 2, (c % 2) * G + i, pl.ds(16, 16)]
                     for c in range(C)]
                )
                bvec = jnp.full((16,), i, jnp.int32)
                plsc.store_scatter(ov, [e_hi0, e_lo0, bvec], acc0)
                plsc.store_scatter(ov, [e_hi1, e_lo0, bvec], acc1)
                return 0

            lax.fori_loop(0, G, seg_body, 0)
            pltpu.async_copy(ov, out_dst(j), osem)

        prefetch(0, idx_a, isem_a)
        prefetch(1, idx_b, isem_b)
        pltpu.async_copy(out_a, out_dst(0), osem_a)
        pltpu.async_copy(out_b, out_dst(1), osem_b)
        start(0, idx_a, idx2_a, rows_a, sem_a, isem_a)

        def pair_body(i, _):
            u = 2 * i
            start(u + 1, idx_b, idx2_b, rows_b, sem_b, isem_b)
            finish(u, idx2_a, rows_a, out_a, sem_a, osem_a)
            start(u + 2, idx_a, idx2_a, rows_a, sem_a, isem_a)
            finish(u + 1, idx2_b, rows_b, out_b, sem_b, osem_b)
            return 0

        lax.fori_loop(0, UNITS_PER_W // 2 - 1, pair_body, 0)
        start(UNITS_PER_W - 1, idx_b, idx2_b, rows_b, sem_b, isem_b)
        finish(UNITS_PER_W - 2, idx2_a, rows_a, out_a, sem_a, osem_a)
        finish(UNITS_PER_W - 1, idx2_b, rows_b, out_b, sem_b, osem_b)
        pltpu.make_async_copy(out_a, out_dst(0), osem_a).wait()
        pltpu.make_async_copy(out_b, out_dst(1), osem_b).wait()
        pltpu.make_async_copy(
            idx_hbm.at[:, pl.ds(NSEG - G, G)], idx_a, isem_a
        ).wait()
        pltpu.make_async_copy(
            idx_hbm.at[:, pl.ds(NSEG - G, G)], idx_b, isem_b
        ).wait()

    return k


_sc_kernel = _make_kernel()


def kernel(mb_t, mtd, W):
    del mtd
    idx = mb_t.astype(jnp.int32).transpose(2, 1, 0).reshape(C, S * B)
    w_st = _w_rowmajor(W.T)
    out5 = _sc_kernel(idx, w_st.reshape(VOCAB_P, EMB))
    return out5.transpose(2, 4, 0, 1, 3).reshape(B, S, EMB)
